# baseline (speedup 1.0000x reference)
import jax
import jax.numpy as jnp
from jax.experimental import pallas as pl
from jax.experimental.pallas import tpu as pltpu

N_ELEM = 16 * 512 * 512
MAX_DIST = 0.3
MAX_OFFSET = 0.04

_ROWS = 4096
_COLS = 1024
_BLK = 512


def _loss_body(m_ref, db_ref, df_ref, ob_ref, of_ref, out_ref):
    @pl.when(pl.program_id(0) == 0)
    def _():
        out_ref[0, 0] = 0.0

    m = m_ref[0]
    db = db_ref[...]
    df = df_ref[...]
    ob = ob_ref[...]
    of = of_ref[...]
    z = 0.0
    t = (jnp.maximum(m - db, z) + jnp.maximum(db - (m + MAX_DIST), z)
         + jnp.maximum(df - m, z) + jnp.maximum(m - MAX_DIST - df, z)
         + jnp.maximum(ob - MAX_OFFSET, z) + jnp.maximum(of - MAX_OFFSET, z)
         + jnp.maximum(ob + MAX_OFFSET, z) + jnp.maximum(of + MAX_OFFSET, z))
    out_ref[0, 0] += jnp.sum(t)


def _loss_from_median(median, db, df, ob, of):
    db2 = db.reshape(_ROWS, _COLS)
    df2 = df.reshape(_ROWS, _COLS)
    ob2 = ob.reshape(_ROWS, _COLS)
    of2 = of.reshape(_ROWS, _COLS)
    m = median.reshape(1).astype(jnp.float32)
    grid = _ROWS // _BLK
    blk = pl.BlockSpec((_BLK, _COLS), lambda i: (i, 0))
    total = pl.pallas_call(
        _loss_body,
        grid=(grid,),
        in_specs=[
            pl.BlockSpec(memory_space=pltpu.SMEM),
            blk, blk, blk, blk,
        ],
        out_specs=pl.BlockSpec((1, 1), lambda i: (0, 0),
                               memory_space=pltpu.SMEM),
        out_shape=jax.ShapeDtypeStruct((1, 1), jnp.float32),
    )(m, db2, df2, ob2, of2)
    return total[0, 0] / jnp.float32(N_ELEM)


def kernel(depth_back, depth_front, offet_back, offet_front):
    sorted_vals = jnp.sort(
        jnp.concatenate((depth_back, depth_front)).flatten())
    is_first = jnp.concatenate(
        (jnp.ones((1,), dtype=bool), sorted_vals[1:] != sorted_vals[:-1]))
    unique_count = jnp.sum(is_first)
    unique_sum = jnp.sum(jnp.where(is_first, sorted_vals, 0.0))
    median = jax.lax.stop_gradient(unique_sum / unique_count)
    return _loss_from_median(median, depth_back, depth_front,
                             offet_back, offet_front)


# speedup vs baseline: 1.2868x; 1.2868x over previous
import jax
import jax.numpy as jnp
from jax import lax
from jax.experimental import pallas as pl
from jax.experimental.pallas import tpu as pltpu
from jax.experimental.pallas import tpu_sc as plsc

MAX_DIST = 0.3
MAX_OFFSET = 0.04

N_ELEM = 16 * 512 * 512
N_KEYS = 2 * N_ELEM
NTILE = 16
NT = N_KEYS // NTILE
C = 8192
VPC = C // 16
NCH1 = NT // C
SB = 2048
SBF = SB + 32
WLREG = (NT // SB + 2) * SBF + C
LOGM = 24
SENT = -1
MAXR = 100
UNROLL = 8

_U = jnp.uint32
SALT_OWN = 0x85EBCA6B


def _canon(kvec):
    u = lax.bitcast_convert_type(kvec, jnp.uint32)
    mag = u & _U(0x7FFFFFFF)
    return jnp.where(mag == _U(0), _U(0), u)


def _salt(r):
    return (_U(0x9E3779B1) + r.astype(jnp.uint32) * _U(0x85EBCA77)) | _U(1)


def _dedup_body(keys, sums, cnts, mapb, wl,
                kbuf0, idxbuf0, sltbuf0, wbuf0, wfbuf0, kwbuf0,
                kbuf1, idxbuf1, sltbuf1, wbuf1, wfbuf1, kwbuf1,
                stage, obuf, cbuf, shcnt, rdbuf, sem0, sem1):
    cid = lax.axis_index("c")
    sid = lax.axis_index("s")
    wid = cid * NTILE + sid
    cid_u = cid.astype(jnp.uint32)
    iota16 = lax.iota(jnp.int32, 16)
    sentvec = jnp.full((16,), SENT, jnp.int32)
    halves = (
        (kbuf0, idxbuf0, sltbuf0, wbuf0, wfbuf0, kwbuf0, sem0),
        (kbuf1, idxbuf1, sltbuf1, wbuf1, wfbuf1, kwbuf1, sem1),
    )

    def prefill_stage():
        for j in range(SBF // 16):
            stage[pl.ds(j * 16, 16)] = sentvec

    def slot_of(u, A, is_valid):
        h = (u * A) >> _U(32 - LOGM)
        slot = (h | (cid_u << _U(LOGM))).astype(jnp.int32)
        return jnp.where(is_valid, slot, SENT)

    prefill_stage()

    A1 = _salt(jnp.int32(0))

    def r1_fill(c, h):
        kbuf, idxbuf, sltbuf = h[0], h[1], h[2]
        base = sid * NT + c * C
        pltpu.sync_copy(keys.at[pl.ds(pl.multiple_of(base, C), C)], kbuf)

        def vec(j, _):
            ds = pl.ds(j * 16, 16)
            u = _canon(kbuf[ds])
            own = ((u * _U(SALT_OWN)) >> _U(31)).astype(jnp.int32)
            valid = own == cid
            sltbuf[ds] = slot_of(u, A1, valid)
            idxbuf[ds] = jnp.where(valid, base + j * 16 + iota16, SENT)
            return 0

        lax.fori_loop(0, VPC, vec, 0, unroll=UNROLL)

    def r1A_pair(p, _):
        descs = []
        for b in range(2):
            h = halves[b]
            r1_fill(p * 2 + b, h)
            descs.append(pltpu.async_copy(
                h[1], mapb.at[plsc.Indices(h[2], ignored_value=SENT)],
                h[6]))
        for d in descs:
            d.wait()
        return 0

    lax.fori_loop(0, NCH1 // 2, r1A_pair, 0)
    plsc.subcore_barrier()

    def vecf_loop(h):
        wbuf, wfbuf = h[3], h[4]
        idxbuf = h[1]

        def vecf(j, _):
            ds = pl.ds(j * 16, 16)
            w = wbuf[ds]
            i = idxbuf[ds]
            lost = (i != SENT) & (w != i)
            wfbuf[ds] = jnp.where(lost, w, SENT)
            return 0

        lax.fori_loop(0, VPC, vecf, 0, unroll=UNROLL)

    def vecr_loop(h, carry):
        kbuf, idxbuf, wbuf, wfbuf, kwbuf = h[0], h[1], h[3], h[4], h[5]

        def vecr(j, car):
            sacc, cacc, cursor, hcur, rcnt, parw = car
            ds = pl.ds(j * 16, 16)
            k = kbuf[ds]
            i = idxbuf[ds]
            w = wbuf[ds]
            wf = wfbuf[ds]
            kw = kwbuf[ds]
            valid = i != SENT
            win = valid & (w == i)
            sacc = sacc + jnp.where(win, k, 0.0)
            cacc = cacc + jnp.where(win, 1, 0)
            surv = (wf != SENT) & (_canon(kw) != _canon(k))
            survi = surv.astype(jnp.int32)
            pos = cursor + plsc.cumsum(survi) - 1
            plsc.store_scatter(stage, [pos], i, mask=surv)
            ns = jnp.sum(survi)
            cursor = cursor + ns
            rcnt = rcnt + ns
            do_flush = cursor >= SB

            @pl.when(do_flush)
            def _():
                stage[pl.ds(cursor, 16)] = sentvec
                pltpu.sync_copy(
                    stage,
                    wl.at[pl.ds(pl.multiple_of(
                        (wid * 2 + parw) * WLREG + hcur, 32), SBF)])
                prefill_stage()

            cursor = jnp.where(do_flush, 0, cursor)
            hcur = jnp.where(do_flush, hcur + SBF, hcur)
            return (sacc, cacc, cursor, hcur, rcnt, parw)

        return lax.fori_loop(0, VPC, vecr, carry, unroll=4)

    def r1B_pair(p, carry):
        dg = []
        for b in range(2):
            h = halves[b]
            r1_fill(p * 2 + b, h)
            dg.append(pltpu.async_copy(
                mapb.at[plsc.Indices(h[2], ignored_value=SENT)], h[3],
                h[6]))
        dk = []
        for b in range(2):
            h = halves[b]
            dg[b].wait()
            vecf_loop(h)
            dk.append(pltpu.async_copy(
                keys.at[plsc.Indices(h[4], ignored_value=SENT)], h[5],
                h[6]))
        for b in range(2):
            h = halves[b]
            dk[b].wait()
            carry = vecr_loop(h, carry)
        return carry

    zero_s = jnp.zeros((16,), jnp.float32)
    zero_c = jnp.zeros((16,), jnp.int32)
    sacc, cacc, cursor, hcur, rcnt, _ = lax.fori_loop(
        0, NCH1 // 2, r1B_pair,
        (zero_s, zero_c, jnp.int32(0), jnp.int32(0), jnp.int32(0),
         jnp.int32(0)))

    def finish_round(parw, cursor, hcur, rcnt):
        @pl.when(cursor > 0)
        def _():
            stage[pl.ds(cursor, 16)] = sentvec
            pltpu.sync_copy(
                stage,
                wl.at[pl.ds(pl.multiple_of(
                    (wid * 2 + parw) * WLREG + hcur, 32), SBF)])
            prefill_stage()

        myn = jnp.where(cursor > 0, hcur + SBF, hcur)
        cbuf[...] = jnp.broadcast_to(rcnt, (16,)).astype(jnp.int32)
        pltpu.sync_copy(
            cbuf, shcnt.at[pl.ds(pl.multiple_of(sid * 16, 16), 16)])
        plsc.subcore_barrier()
        pltpu.sync_copy(shcnt, rdbuf)
        mx = rdbuf[pl.ds(0, 16)]

        def rmax(jj, m):
            return jnp.maximum(m, rdbuf[pl.ds(jj * 16, 16)])

        mx = lax.fori_loop(1, NTILE, rmax, mx)
        go = jnp.max(mx, axis=0) > 0
        plsc.subcore_barrier()
        return myn, go

    myn, go = finish_round(jnp.int32(0), cursor, hcur, rcnt)

    h0 = halves[0]

    def cond(st):
        r, par, myn, go, sacc, cacc = st
        return go & (r < MAXR)

    def body(st):
        r, par, myn, sacc, cacc = st[0], st[1], st[2], st[4], st[5]
        A = _salt(r)
        nch = (myn + (C - 1)) // C
        kbuf, idxbuf, sltbuf, wfbuf = h0[0], h0[1], h0[2], h0[4]
        sem = h0[6]

        def fill_bufs(c):
            base = c * C
            pltpu.sync_copy(
                wl.at[pl.ds(pl.multiple_of(
                    (wid * 2 + par) * WLREG + base, 32), C)], wfbuf)

            def vsan(j, _):
                ds = pl.ds(j * 16, 16)
                i = wfbuf[ds]
                pos = base + j * 16 + iota16
                v = (pos < myn) & (i != SENT)
                idxbuf[ds] = jnp.where(v, i, SENT)
                return 0

            lax.fori_loop(0, VPC, vsan, 0, unroll=UNROLL)
            pltpu.async_copy(
                keys.at[plsc.Indices(idxbuf, ignored_value=SENT)], kbuf,
                sem).wait()

            def vslt(j, _):
                ds = pl.ds(j * 16, 16)
                i = idxbuf[ds]
                u = _canon(kbuf[ds])
                sltbuf[ds] = slot_of(u, A, i != SENT)
                return 0

            lax.fori_loop(0, VPC, vslt, 0, unroll=UNROLL)

        def Achunk(c, _):
            fill_bufs(c)
            pltpu.async_copy(
                idxbuf, mapb.at[plsc.Indices(sltbuf, ignored_value=SENT)],
                sem).wait()
            return 0

        lax.fori_loop(0, nch, Achunk, 0)
        plsc.subcore_barrier()

        def Bchunk(c, carry):
            fill_bufs(c)
            pltpu.async_copy(
                mapb.at[plsc.Indices(sltbuf, ignored_value=SENT)], h0[3],
                sem).wait()
            vecf_loop(h0)
            pltpu.async_copy(
                keys.at[plsc.Indices(h0[4], ignored_value=SENT)], h0[5],
                sem).wait()
            return vecr_loop(h0, carry)

        sacc2, cacc2, cursor, hcur, rcnt, _ = lax.fori_loop(
            0, nch, Bchunk,
            (sacc, cacc, jnp.int32(0), jnp.int32(0), jnp.int32(0), 1 - par))
        myn2, go2 = finish_round(1 - par, cursor, hcur, rcnt)
        return (r + 1, 1 - par, myn2, go2, sacc2, cacc2)

    st = lax.while_loop(
        cond, body, (jnp.int32(1), jnp.int32(0), myn, go, sacc, cacc))
    sacc, cacc = st[4], st[5]

    obuf[...] = sacc
    pltpu.sync_copy(obuf, sums.at[pl.ds(pl.multiple_of(wid * 16, 16), 16)])
    cbuf[...] = cacc
    pltpu.sync_copy(cbuf, cnts.at[pl.ds(pl.multiple_of(wid * 16, 16), 16)])


def _sc_unique_partials(keys):
    mesh = plsc.VectorSubcoreMesh(core_axis_name="c", subcore_axis_name="s")
    vb = [
        pltpu.VMEM((C,), jnp.float32),
        pltpu.VMEM((C,), jnp.int32),
        pltpu.VMEM((C,), jnp.int32),
        pltpu.VMEM((C,), jnp.int32),
        pltpu.VMEM((C,), jnp.int32),
        pltpu.VMEM((C,), jnp.float32),
    ]
    f = pl.kernel(
        _dedup_body,
        compiler_params=pltpu.CompilerParams(needs_layout_passes=False),
        out_type=[
            jax.ShapeDtypeStruct((512,), jnp.float32),
            jax.ShapeDtypeStruct((512,), jnp.int32),
            jax.ShapeDtypeStruct((1 << (LOGM + 1),), jnp.int32),
            jax.ShapeDtypeStruct((64 * WLREG,), jnp.int32),
        ],
        mesh=mesh,
        scratch_types=vb + vb + [
            pltpu.VMEM((SBF,), jnp.int32),
            pltpu.VMEM((16,), jnp.float32),
            pltpu.VMEM((16,), jnp.int32),
            pltpu.VMEM_SHARED((256,), jnp.int32),
            pltpu.VMEM((256,), jnp.int32),
            pltpu.SemaphoreType.DMA,
            pltpu.SemaphoreType.DMA,
        ],
    )
    sums, cnts, _, _ = f(keys)
    return sums, cnts


_ROWS = 4096
_COLS = 1024
_BLK = 1024


def _loss_body(ps_ref, pc_ref, db_ref, df_ref, ob_ref, of_ref, out_ref):
    @pl.when(pl.program_id(0) == 0)
    def _():
        out_ref[0, 0] = 0.0

    m = jnp.sum(ps_ref[...]) / jnp.sum(pc_ref[...].astype(jnp.float32))
    db = db_ref[...]
    df = df_ref[...]
    ob = ob_ref[...]
    of = of_ref[...]
    z = 0.0
    t = (jnp.maximum(m - db, z) + jnp.maximum(db - (m + MAX_DIST), z)
         + jnp.maximum(df - m, z) + jnp.maximum(m - MAX_DIST - df, z)
         + jnp.maximum(ob - MAX_OFFSET, z) + jnp.maximum(of - MAX_OFFSET, z)
         + jnp.maximum(ob + MAX_OFFSET, z) + jnp.maximum(of + MAX_OFFSET, z))
    out_ref[0, 0] += jnp.sum(t)


def _loss_from_partials(sums, cnts, db, df, ob, of):
    db2 = db.reshape(_ROWS, _COLS)
    df2 = df.reshape(_ROWS, _COLS)
    ob2 = ob.reshape(_ROWS, _COLS)
    of2 = of.reshape(_ROWS, _COLS)
    ps = sums.reshape(4, 128)
    pc = cnts.reshape(4, 128)
    grid = _ROWS // _BLK
    blk = pl.BlockSpec((_BLK, _COLS), lambda i: (i, 0))
    pspec = pl.BlockSpec((4, 128), lambda i: (0, 0))
    total = pl.pallas_call(
        _loss_body,
        grid=(grid,),
        in_specs=[pspec, pspec, blk, blk, blk, blk],
        out_specs=pl.BlockSpec((1, 1), lambda i: (0, 0),
                               memory_space=pltpu.SMEM),
        out_shape=jax.ShapeDtypeStruct((1, 1), jnp.float32),
    )(ps, pc, db2, df2, ob2, of2)
    return total[0, 0] / jnp.float32(N_ELEM)


def kernel(depth_back, depth_front, offet_back, offet_front):
    keys = jnp.concatenate(
        (depth_back.reshape(-1), depth_front.reshape(-1)))
    sums, cnts = _sc_unique_partials(keys)
    return _loss_from_partials(sums, cnts, depth_back, depth_front,
                               offet_back, offet_front)
